# Initial kernel scaffold; baseline (speedup 1.0000x reference)
import jax
import jax.numpy as jnp
from jax.experimental import pallas as pl


def kernel(x, pos, batch, params):
    raise NotImplementedError("write your pallas kernel here")


# speedup vs baseline: 1.0025x; 1.0025x over previous
import functools
import numpy as np
import jax
import jax.numpy as jnp
from jax.experimental import pallas as pl
from jax.experimental.pallas import tpu as pltpu

B = 16
N_PER = 2048
NUM_CLASSES = 13
SCALING = 2.0


def _lora_linear(p, x):
    return x @ p['W'].T + p['b'] + (x @ p['A'].T) @ p['Bm'].T * SCALING


def _mlp(ps, x):
    for i, p in enumerate(ps):
        x = _lora_linear(p, x)
        if i < len(ps) - 1:
            x = jax.nn.relu(x)
    return x


def _fps(pos, m):
    n = pos.shape[1]

    def per_cloud(p):
        def body(i, state):
            dmin, idxs, last = state
            dist = jnp.sum((p - p[last]) ** 2, axis=-1)
            dmin = jnp.minimum(dmin, dist)
            nxt = jnp.argmax(dmin).astype(jnp.int32)
            idxs = idxs.at[i].set(nxt)
            return (dmin, idxs, nxt)

        dmin0 = jnp.full((n,), jnp.inf, jnp.float32)
        idxs0 = jnp.zeros((m,), jnp.int32)
        _, idxs, _ = jax.lax.fori_loop(1, m, body, (dmin0, idxs0, jnp.int32(0)))
        return idxs

    return jax.vmap(per_cloud)(pos)


def _radius_neighbors(pos, idx, r, k):
    cpos = jnp.take_along_axis(pos, idx[..., None], axis=1)
    d2 = jnp.sum((cpos[:, :, None, :] - pos[:, None, :, :]) ** 2, axis=-1)
    n = pos.shape[1]
    order = jnp.where(d2 <= r * r, jnp.arange(n, dtype=jnp.int32)[None, None, :], n)
    vals, nbr = jax.lax.top_k(-order, k)
    valid = (-vals) < n
    nbr = jnp.where(valid, nbr, 0)
    return cpos, nbr, valid


def _gather_points(t, nbr):
    b, m, k = nbr.shape
    flat = jnp.take_along_axis(t, nbr.reshape(b, m * k)[..., None], axis=1)
    return flat.reshape(b, m, k, t.shape[-1])


def _sa_module(mlp_ps, x, pos, ratio, r):
    m = int(np.ceil(ratio * pos.shape[1]))
    idx = _fps(jax.lax.stop_gradient(pos), m)
    cpos, nbr, valid = _radius_neighbors(pos, idx, r, 64)
    xj = _gather_points(x, nbr)
    pj = _gather_points(pos, nbr)
    msg = jnp.concatenate([xj, pj - cpos[:, :, None, :]], axis=-1)
    h = _mlp(mlp_ps, msg)
    h = jnp.where(valid[..., None], h, -jnp.inf)
    h = jnp.max(h, axis=2)
    h = jnp.where(jnp.isfinite(h), h, 0.0)
    return h, cpos


def _knn_interpolate(x, pos_src, pos_dst, k):
    d2 = jnp.sum((pos_dst[:, :, None, :] - pos_src[:, None, :, :]) ** 2, axis=-1)
    negd, idx = jax.lax.top_k(-d2, k)
    w = 1.0 / jnp.clip(-negd, 1e-16)
    xg = _gather_points(x, idx)
    return jnp.sum(w[..., None] * xg, axis=2) / jnp.sum(w, axis=2)[..., None]


def _head_kernel(x_ref, w1_ref, b1_ref, w2_ref, b2_ref, w3_ref, b3_ref, o_ref):
    x = x_ref[...]
    h = jnp.maximum(x @ w1_ref[...] + b1_ref[...], 0.0)
    h = jnp.maximum(h @ w2_ref[...] + b2_ref[...], 0.0)
    o = h @ w3_ref[...] + b3_ref[...]
    mx = jnp.max(o, axis=-1, keepdims=True)
    e = jnp.where(o > -1e30, jnp.exp(o - mx), 0.0)
    lse = jnp.log(jnp.sum(e, axis=-1, keepdims=True)) + mx
    o_ref[...] = o - lse


def _fold(p):
    return p['W'] + SCALING * (p['Bm'] @ p['A'])


def _head_pallas(ps, x):
    n, _ = x.shape
    w1 = _fold(ps[0]).T
    w2 = _fold(ps[1]).T
    w3f = _fold(ps[2]).T
    w3 = jnp.zeros((128, 128), jnp.float32).at[:, :13].set(w3f)
    b3 = jnp.full((128,), -jnp.inf, jnp.float32).at[:13].set(ps[2]['b'])
    grid = n // 2048
    out = pl.pallas_call(
        _head_kernel,
        grid=(grid,),
        in_specs=[
            pl.BlockSpec((2048, 128), lambda i: (i, 0)),
            pl.BlockSpec((128, 128), lambda i: (0, 0)),
            pl.BlockSpec((128,), lambda i: (0,)),
            pl.BlockSpec((128, 128), lambda i: (0, 0)),
            pl.BlockSpec((128,), lambda i: (0,)),
            pl.BlockSpec((128, 128), lambda i: (0, 0)),
            pl.BlockSpec((128,), lambda i: (0,)),
        ],
        out_specs=pl.BlockSpec((2048, 128), lambda i: (i, 0)),
        out_shape=jax.ShapeDtypeStruct((n, 128), jnp.float32),
    )(x, w1, ps[0]['b'], w2, ps[1]['b'], w3, b3)
    return out[:, :13]


def kernel(x, pos, batch, params):
    x0 = x.reshape(B, N_PER, -1)
    p0 = pos.reshape(B, N_PER, 3)
    x1, p1 = _sa_module(params['sa1'], x0, p0, 0.2, 0.2)
    x2, p2 = _sa_module(params['sa2'], x1, p1, 0.25, 0.4)
    g = _mlp(params['sa3'], jnp.concatenate([x2, p2], axis=-1))
    x3 = jnp.max(g, axis=1)
    h3 = jnp.broadcast_to(x3[:, None, :], (B, x2.shape[1], x3.shape[-1]))
    h3 = _mlp(params['fp3'], jnp.concatenate([h3, x2], axis=-1))
    h2 = _knn_interpolate(h3, p2, p1, 3)
    h2 = _mlp(params['fp2'], jnp.concatenate([h2, x1], axis=-1))
    h1 = _knn_interpolate(h2, p1, p0, 3)
    h1 = _mlp(params['fp1'], jnp.concatenate([h1, x0], axis=-1))
    out = _head_pallas(params['head'], h1.reshape(B * N_PER, -1))
    return out.reshape(B * N_PER, NUM_CLASSES)


# speedup vs baseline: 10.7187x; 10.6916x over previous
import functools
import numpy as np
import jax
import jax.numpy as jnp
from jax import lax
from jax.experimental import pallas as pl
from jax.experimental.pallas import tpu as pltpu
from jax.experimental.pallas import tpu_sc as plsc

B = 16
N0 = 2048
NUM_CLASSES = 13
SCALING = 2.0

M1 = 410
M1P = 512
M2 = 103
M2P = 128
K = 64
R1 = 0.2
R2 = 0.4
BIG = 1e30
PPAD = 1e9

F32 = jnp.float32
I32 = jnp.int32


def _iota(shape, dim):
    return lax.broadcasted_iota(I32, shape, dim)


def _fiota(shape, dim):
    return lax.broadcasted_iota(F32, shape, dim)


def _fold(wt, at_, bmt):
    return wt + SCALING * jnp.dot(at_, bmt, preferred_element_type=F32)


def _fps_kernel(n_real, m, pos3_ref, out_ref):
    pos3 = pos3_ref[...]
    n = pos3.shape[2]
    jn = _iota((B, n), 1)
    dmin0 = jnp.where(jn < n_real, jnp.full((B, n), jnp.inf, F32),
                      jnp.full((B, n), -jnp.inf, F32))
    idxs0 = jnp.zeros((B, out_ref.shape[1]), I32)
    last0 = pos3[:, :, 0:1]

    def body(i, state):
        dmin, idxs, lastp = state
        diff = pos3 - lastp
        dist = jnp.sum(diff * diff, axis=1)
        dmin = jnp.minimum(dmin, dist)
        mx = jnp.max(dmin, axis=1, keepdims=True)
        eq = dmin == mx
        nxt = jnp.min(jnp.where(eq, jn, n), axis=1, keepdims=True)
        idxs = jnp.where(_iota(idxs.shape, 1) == i, nxt, idxs)
        oh = (jn == nxt).astype(F32)
        lastp = jnp.sum(pos3 * oh[:, None, :], axis=2, keepdims=True)
        return dmin, idxs, lastp

    _, idxs, _ = lax.fori_loop(1, m, body, (dmin0, idxs0, last0))
    out_ref[...] = idxs


def _fps(pos3, n_real, m, mpad):
    npad = pos3.shape[2]
    return pl.pallas_call(
        functools.partial(_fps_kernel, n_real, m),
        in_specs=[pl.BlockSpec((B, 3, npad), lambda: (0, 0, 0))],
        out_specs=pl.BlockSpec((B, mpad), lambda: (0, 0)),
        out_shape=jax.ShapeDtypeStruct((B, mpad), I32),
    )(pos3)


def _nbr_kernel(n_real, r2, stride, pos3_ref, posnd_ref, idx_ref,
                nbr_ref, val_ref, cpos_ref):
    b = pl.program_id(0)
    pos3 = pos3_ref[0]
    posnd = posnd_ref[0]
    idx = idx_ref[0]
    mp = idx.shape[0]
    npad = pos3.shape[1]
    nb = npad // 128

    ohm = (_iota((mp, npad), 1) == idx).astype(F32)
    cpos = jnp.dot(ohm, posnd, preferred_element_type=F32)
    d2 = jnp.zeros((mp, npad), F32)
    for c in range(3):
        diff = cpos[:, c:c + 1] - pos3[c:c + 1, :]
        d2 = d2 + diff * diff
    maskf = (d2 <= r2).astype(F32)

    mask3 = maskf.reshape(mp, nb, 128)
    li = _iota((128, 128), 0)
    lj = _iota((128, 128), 1)
    tri_inc = (li <= lj).astype(F32)
    intra = jnp.dot(maskf.reshape(mp * nb, 128), tri_inc,
                    preferred_element_type=F32).reshape(mp, nb, 128)
    bsum = jnp.sum(mask3, axis=2)
    bi = _iota((nb, nb), 0)
    bj = _iota((nb, nb), 1)
    tri_exc = (bi < bj).astype(F32)
    base = jnp.dot(bsum, tri_exc, preferred_element_type=F32)
    crank = intra + base[:, :, None]

    cnt = jnp.sum(bsum, axis=1, keepdims=True)
    nbrf = jnp.zeros((mp, K), F32)
    tcol = _iota((mp, K), 1)
    for t in range(K):
        ind = (crank <= float(t)).astype(F32)
        c_t = jnp.sum(jnp.sum(ind, axis=2), axis=1, keepdims=True)
        nbrf = jnp.where(tcol == t, c_t, nbrf)
    nbrf = jnp.minimum(nbrf, float(n_real - 1))
    validf = (tcol.astype(F32) < jnp.minimum(cnt, float(K))).astype(F32)

    nbr_ref[0] = nbrf.astype(I32) + b * stride
    val_ref[0] = validf
    cpos_ref[0] = cpos


def _nbr(pos3, posnd, idx3, n_real, r, mpad, stride):
    npad = pos3.shape[2]
    return pl.pallas_call(
        functools.partial(_nbr_kernel, n_real, r * r, stride),
        grid=(B,),
        in_specs=[
            pl.BlockSpec((1, 3, npad), lambda b: (b, 0, 0)),
            pl.BlockSpec((1, npad, 4), lambda b: (b, 0, 0)),
            pl.BlockSpec((1, mpad, 1), lambda b: (b, 0, 0)),
        ],
        out_specs=[
            pl.BlockSpec((1, mpad, K), lambda b: (b, 0, 0)),
            pl.BlockSpec((1, mpad, K), lambda b: (b, 0, 0)),
            pl.BlockSpec((1, mpad, 4), lambda b: (b, 0, 0)),
        ],
        out_shape=[
            jax.ShapeDtypeStruct((B, mpad, K), I32),
            jax.ShapeDtypeStruct((B, mpad, K), F32),
            jax.ShapeDtypeStruct((B, mpad, 4), F32),
        ],
    )(pos3, posnd, idx3)


def _sc_gather(table, idx, group=4):
    e = idx.shape[0]
    d = table.shape[1]
    nw = 32
    pw128 = e // (nw * 128)
    nloop = pw128 // group
    idx2 = idx.reshape(e // 128, 128)
    mesh = plsc.VectorSubcoreMesh(core_axis_name="c", subcore_axis_name="s")

    @functools.partial(
        pl.kernel,
        out_type=jax.ShapeDtypeStruct((e, d), F32),
        mesh=mesh,
        scratch_types=[
            pltpu.VMEM((group, 128), I32),
            pltpu.VMEM((group * 128, d), F32),
            pltpu.SemaphoreType.DMA,
        ],
    )
    def k(table_hbm, idx_hbm, out_hbm, idx_v, rows_v, sem):
        wid = lax.axis_index("s") * 2 + lax.axis_index("c")
        row0 = wid * pw128

        def body(g, carry):
            r0 = row0 + g * group
            pltpu.sync_copy(idx_hbm.at[pl.ds(r0, group)], idx_v)
            copies = []
            for j in range(group):
                copies.append(pltpu.async_copy(
                    table_hbm.at[idx_v.at[j]],
                    rows_v.at[pl.ds(j * 128, 128)], sem))
            for cp in copies:
                cp.wait()
            pltpu.sync_copy(rows_v, out_hbm.at[pl.ds(r0 * 128, group * 128)])
            return carry

        lax.fori_loop(0, nloop, body, 0)

    return k(table, idx2)


def _pre_kernel(in_ref, w_ref, a_ref, p_ref, b_ref, out_ref):
    wc = _fold(w_ref[...], a_ref[...], p_ref[...])
    out_ref[0] = (jnp.dot(in_ref[0], wc, preferred_element_type=F32)
                  + b_ref[...])


def _pre_mlp(inp, wp):
    _, np_, d = inp.shape

    def wspec(shape):
        return pl.BlockSpec(shape, lambda b: tuple(0 for _ in shape))

    return pl.pallas_call(
        _pre_kernel,
        grid=(B,),
        in_specs=[pl.BlockSpec((1, np_, d), lambda b: (b, 0, 0)),
                  wspec((d, 128)), wspec((d, 8)), wspec((8, 128)),
                  wspec((1, 128))],
        out_specs=pl.BlockSpec((1, np_, 128), lambda b: (b, 0, 0)),
        out_shape=jax.ShapeDtypeStruct((B, np_, 128), F32),
    )(inp, *wp)


def _edge_kernel(g_ref, cp_ref, val_ref,
                 wp_ref, ap_ref, pm_ref,
                 w2_ref, a2_ref, p2_ref, b2_ref,
                 w3_ref, a3_ref, p3_ref, b3_ref, out_ref):
    g2 = g_ref[0]
    cp = cp_ref[0]
    vmask = val_ref[0]
    mc = cp.shape[0]
    c3 = out_ref.shape[2]

    w1p = _fold(wp_ref[...], ap_ref[...], pm_ref[...])
    w2c = _fold(w2_ref[...], a2_ref[...], p2_ref[...])
    w3c = _fold(w3_ref[...], a3_ref[...], p3_ref[...])

    ccon = jnp.dot(cp, w1p, preferred_element_type=F32)
    h = g2.reshape(mc, K, 128) - ccon[:, None, :]
    h = jnp.maximum(h, 0.0).reshape(mc * K, 128)
    h = jnp.maximum(jnp.dot(h, w2c, preferred_element_type=F32)
                    + b2_ref[...], 0.0)
    h = jnp.dot(h, w3c, preferred_element_type=F32) + b3_ref[...]
    h = h.reshape(mc, K, c3)
    h = jnp.where(vmask[:, :, None] > 0.0, h, -BIG)
    mx = jnp.max(h, axis=1)
    out_ref[0] = jnp.where(mx > -BIG * 0.5, mx, 0.0)


def _edge_mlp(g3, cpos, validf, wp, couts, mpad, mc):
    grid_m = mpad // mc
    c2, c3 = couts

    def wspec(shape):
        return pl.BlockSpec(shape, lambda b, i: tuple(0 for _ in shape))

    return pl.pallas_call(
        _edge_kernel,
        grid=(B, grid_m),
        in_specs=[
            pl.BlockSpec((1, mc * K, 128), lambda b, i: (b, i, 0)),
            pl.BlockSpec((1, mc, 4), lambda b, i: (b, i, 0)),
            pl.BlockSpec((1, mc, K), lambda b, i: (b, i, 0)),
            wspec((4, 128)), wspec((4, 8)), wspec((8, 128)),
            wspec((128, c2)), wspec((128, 8)), wspec((8, c2)), wspec((1, c2)),
            wspec((c2, c3)), wspec((c2, 8)), wspec((8, c3)), wspec((1, c3)),
        ],
        out_specs=pl.BlockSpec((1, mc, c3), lambda b, i: (b, i, 0)),
        out_shape=jax.ShapeDtypeStruct((B, mpad, c3), F32),
    )(g3, cpos, validf, *wp)


def _sa3_kernel(w_refs, in_ref, out_ref):
    (w1, a1, p1, b1, w2, a2, p2, b2, w3, a3, p3, b3) = w_refs
    x = in_ref[0]
    h = jnp.maximum(jnp.dot(x, _fold(w1[...], a1[...], p1[...]),
                            preferred_element_type=F32) + b1[...], 0.0)
    h = jnp.maximum(jnp.dot(h, _fold(w2[...], a2[...], p2[...]),
                            preferred_element_type=F32) + b2[...], 0.0)
    h = jnp.dot(h, _fold(w3[...], a3[...], p3[...]),
                preferred_element_type=F32) + b3[...]
    h = jnp.where(_iota(h.shape, 0) < M2, h, -BIG)
    out_ref[0] = jnp.max(h, axis=0, keepdims=True)


def _sa3(in2, wp):
    cin = in2.shape[2]

    def body(*refs):
        _sa3_kernel(refs[1:13], refs[0], refs[13])

    def wspec(shape):
        return pl.BlockSpec(shape, lambda b: tuple(0 for _ in shape))

    shapes = [(cin, 256), (cin, 8), (8, 256), (1, 256),
              (256, 512), (256, 8), (8, 512), (1, 512),
              (512, 1024), (512, 8), (8, 1024), (1, 1024)]
    return pl.pallas_call(
        body,
        grid=(B,),
        in_specs=[pl.BlockSpec((1, M2P, cin), lambda b: (b, 0, 0))]
        + [wspec(s) for s in shapes],
        out_specs=pl.BlockSpec((1, 1, 1024), lambda b: (b, 0, 0)),
        out_shape=jax.ShapeDtypeStruct((B, 1, 1024), F32),
    )(in2, *wp)


def _fp3_kernel(x3_ref, x2_ref, w1a_ref, a1a_ref, w1b_ref, a1b_ref,
                p1_ref, b1_ref, w2_ref, a2_ref, p2_ref, b2_ref, out_ref):
    x3 = x3_ref[0]
    x2 = x2_ref[0]
    w1a = _fold(w1a_ref[...], a1a_ref[...], p1_ref[...])
    row = jnp.dot(x3, w1a, preferred_element_type=F32) + b1_ref[...]
    h = jnp.maximum(jnp.dot(x2, _fold(w1b_ref[...], a1b_ref[...], p1_ref[...]),
                            preferred_element_type=F32) + row, 0.0)
    w2c = _fold(w2_ref[...], a2_ref[...], p2_ref[...])
    out_ref[0] = jnp.dot(h, w2c, preferred_element_type=F32) + b2_ref[...]


def _fp3(x3, x2, wp):
    def wspec(shape):
        return pl.BlockSpec(shape, lambda b: tuple(0 for _ in shape))

    shapes = [(1024, 256), (1024, 8), (256, 256), (256, 8), (8, 256), (1, 256),
              (256, 256), (256, 8), (8, 256), (1, 256)]
    return pl.pallas_call(
        _fp3_kernel,
        grid=(B,),
        in_specs=[pl.BlockSpec((1, 1, 1024), lambda b: (b, 0, 0)),
                  pl.BlockSpec((1, M2P, 256), lambda b: (b, 0, 0))]
        + [wspec(s) for s in shapes],
        out_specs=pl.BlockSpec((1, M2P, 256), lambda b: (b, 0, 0)),
        out_shape=jax.ShapeDtypeStruct((B, M2P, 256), F32),
    )(x3, x2, *wp)


def _interp_weights(pdst, psrct, n_src_real):
    nd = pdst.shape[0]
    nsp = psrct.shape[1]
    d2 = jnp.zeros((nd, nsp), F32)
    for c in range(3):
        diff = pdst[:, c:c + 1] - psrct[c:c + 1, :]
        d2 = d2 + diff * diff
    jcol = _iota((nd, nsp), 1)
    d2m = jnp.where(jcol < n_src_real, d2, BIG)
    wacc = jnp.zeros((nd, nsp), F32)
    sel = jnp.zeros((nd, nsp), F32)
    for _ in range(3):
        cur = jnp.where(sel > 0.0, BIG, d2m)
        v = jnp.min(cur, axis=1, keepdims=True)
        eq = cur == v
        jm = jnp.min(jnp.where(eq, jcol, nsp), axis=1, keepdims=True)
        oh = (jcol == jm).astype(F32)
        w = 1.0 / jnp.maximum(v, 1e-16)
        wacc = wacc + oh * w
        sel = sel + oh
    return wacc / jnp.sum(wacc, axis=1, keepdims=True)


def _fp2_kernel(pdst_ref, psrct_ref, h3_ref, x1_ref,
                w1a_ref, a1a_ref, w1b_ref, a1b_ref, p1_ref, b1_ref,
                w2_ref, a2_ref, p2_ref, b2_ref, out_ref):
    wn = _interp_weights(pdst_ref[0], psrct_ref[0], M2)
    interp = jnp.dot(wn, h3_ref[0], preferred_element_type=F32)
    h = (jnp.dot(interp, _fold(w1a_ref[...], a1a_ref[...], p1_ref[...]),
                 preferred_element_type=F32)
         + jnp.dot(x1_ref[0], _fold(w1b_ref[...], a1b_ref[...], p1_ref[...]),
                   preferred_element_type=F32) + b1_ref[...])
    h = jnp.maximum(h, 0.0)
    out_ref[0] = (jnp.dot(h, _fold(w2_ref[...], a2_ref[...], p2_ref[...]),
                          preferred_element_type=F32) + b2_ref[...])


def _fp2(cpos1, psrc2t, h3, x1, wp):
    def wspec(shape):
        return pl.BlockSpec(shape, lambda b: tuple(0 for _ in shape))

    shapes = [(256, 256), (256, 8), (128, 256), (128, 8), (8, 256), (1, 256),
              (256, 128), (256, 8), (8, 128), (1, 128)]
    return pl.pallas_call(
        _fp2_kernel,
        grid=(B,),
        in_specs=[pl.BlockSpec((1, M1P, 4), lambda b: (b, 0, 0)),
                  pl.BlockSpec((1, 4, M2P), lambda b: (b, 0, 0)),
                  pl.BlockSpec((1, M2P, 256), lambda b: (b, 0, 0)),
                  pl.BlockSpec((1, M1P, 128), lambda b: (b, 0, 0))]
        + [wspec(s) for s in shapes],
        out_specs=pl.BlockSpec((1, M1P, 128), lambda b: (b, 0, 0)),
        out_shape=jax.ShapeDtypeStruct((B, M1P, 128), F32),
    )(cpos1, psrc2t, h3, x1, *wp)


def _fp1_head_kernel(p0_ref, psrc1t_ref, h2_ref, x0_ref, wrefs, out_ref):
    (w1a, a1a, w1b, a1b, p1, b1, w2, a2, p2, b2, w3, a3, p3, b3,
     hw1, ha1, hp1, hb1, hw2, ha2, hp2, hb2, hw3, ha3, hp3, hb3) = wrefs
    wn = _interp_weights(p0_ref[0], psrc1t_ref[0], M1)
    interp = jnp.dot(wn, h2_ref[0], preferred_element_type=F32)
    h = (jnp.dot(interp, _fold(w1a[...], a1a[...], p1[...]),
                 preferred_element_type=F32)
         + jnp.dot(x0_ref[0], _fold(w1b[...], a1b[...], p1[...]),
                   preferred_element_type=F32) + b1[...])
    h = jnp.maximum(h, 0.0)
    h = jnp.maximum(jnp.dot(h, _fold(w2[...], a2[...], p2[...]),
                            preferred_element_type=F32) + b2[...], 0.0)
    h = jnp.dot(h, _fold(w3[...], a3[...], p3[...]),
                preferred_element_type=F32) + b3[...]
    h = jnp.maximum(jnp.dot(h, _fold(hw1[...], ha1[...], hp1[...]),
                            preferred_element_type=F32) + hb1[...], 0.0)
    h = jnp.maximum(jnp.dot(h, _fold(hw2[...], ha2[...], hp2[...]),
                            preferred_element_type=F32) + hb2[...], 0.0)
    o = jnp.dot(h, _fold(hw3[...], ha3[...], hp3[...]),
                preferred_element_type=F32) + hb3[...]
    mx = jnp.max(o, axis=-1, keepdims=True)
    e = jnp.where(o > -BIG * 0.5, jnp.exp(o - mx), 0.0)
    lse = jnp.log(jnp.sum(e, axis=-1, keepdims=True)) + mx
    out_ref[0] = o - lse


def _fp1_head(p0nd, psrc1t, h2, x0e, wp):
    def wspec(shape):
        return pl.BlockSpec(shape, lambda b: tuple(0 for _ in shape))

    shapes = [(128, 128), (128, 8), (8, 128), (8, 8), (8, 128), (1, 128),
              (128, 128), (128, 8), (8, 128), (1, 128),
              (128, 128), (128, 8), (8, 128), (1, 128),
              (128, 128), (128, 8), (8, 128), (1, 128),
              (128, 128), (128, 8), (8, 128), (1, 128),
              (128, 128), (128, 8), (8, 128), (1, 128)]

    def body(*refs):
        _fp1_head_kernel(refs[0], refs[1], refs[2], refs[3],
                         refs[4:30], refs[30])

    return pl.pallas_call(
        body,
        grid=(B,),
        in_specs=[pl.BlockSpec((1, N0, 4), lambda b: (b, 0, 0)),
                  pl.BlockSpec((1, 4, M1P), lambda b: (b, 0, 0)),
                  pl.BlockSpec((1, M1P, 128), lambda b: (b, 0, 0)),
                  pl.BlockSpec((1, N0, 8), lambda b: (b, 0, 0))]
        + [wspec(s) for s in shapes],
        out_specs=pl.BlockSpec((1, N0, 128), lambda b: (b, 0, 0)),
        out_shape=jax.ShapeDtypeStruct((B, N0, 128), F32),
    )(p0nd, psrc1t, h2, x0e, *wp)


def _prep_layer(p, fin_pad, split=None):
    w, a, bm, bias = p['W'], p['A'], p['Bm'], p['b']
    fout, fin = w.shape
    wt = w.T
    at = a.T
    bmt = bm.T
    b2 = bias[None, :]
    if split is None:
        pad = fin_pad - fin
        if pad:
            wt = jnp.pad(wt, ((0, pad), (0, 0)))
            at = jnp.pad(at, ((0, pad), (0, 0)))
        return [wt, at, bmt, b2]
    s0 = split
    wta, wtb = wt[:s0], wt[s0:]
    ata, atb = at[:s0], at[s0:]
    padb = fin_pad - (fin - s0)
    if padb:
        wtb = jnp.pad(wtb, ((0, padb), (0, 0)))
        atb = jnp.pad(atb, ((0, padb), (0, 0)))
    return [wta, ata, wtb, atb, bmt, b2]


def _pad_cols(arrs, cout):
    wt, at, bmt, b2 = arrs
    pad = cout - wt.shape[1]
    if pad:
        wt = jnp.pad(wt, ((0, 0), (0, pad)))
        bmt = jnp.pad(bmt, ((0, 0), (0, pad)))
        b2 = jnp.pad(b2, ((0, 0), (0, pad)))
    return [wt, at, bmt, b2]


def _prep_head_last(p):
    w, a, bm, bias = p['W'], p['A'], p['Bm'], p['b']
    wt = jnp.pad(w.T, ((0, 0), (0, 128 - NUM_CLASSES)))
    bmt = jnp.pad(bm.T, ((0, 0), (0, 128 - NUM_CLASSES)))
    b2 = jnp.full((1, 128), -jnp.inf, F32).at[0, :NUM_CLASSES].set(bias)
    return [wt, a.T, bmt, b2]


def kernel(x, pos, batch, params):
    x0 = x.reshape(B, N0, 6)
    p0 = pos.reshape(B, N0, 3)
    p0t = jnp.transpose(p0, (0, 2, 1))
    p0nd = jnp.pad(p0, ((0, 0), (0, 0), (0, 1)))

    idx1 = _fps(p0t, N0, M1, M1P)
    nbr1, val1, cpos1 = _nbr(p0t, p0nd, idx1[..., None], N0, R1, M1P, N0)
    in1 = jnp.concatenate(
        [x0, p0, jnp.zeros((B, N0, 7), F32)], axis=2)
    l1a = _pad_cols(_prep_layer(params['sa1'][0], 16), 128)
    u1 = _pre_mlp(in1, l1a)
    g1 = _sc_gather(u1.reshape(B * N0, 128), nbr1.reshape(-1))
    g1 = g1.reshape(B, M1P * K, 128)
    wp1 = ([l1a[0][6:10], l1a[1][6:10], l1a[2]]
           + _prep_layer(params['sa1'][1], 128)
           + _prep_layer(params['sa1'][2], 64))
    x1 = _edge_mlp(g1, cpos1, val1, wp1, (64, 128), M1P, 128)

    p1 = cpos1[..., :3]
    jrow = jnp.arange(M1P)[None, :, None]
    p1m = jnp.where(jrow < M1, p1, PPAD)
    p1t = jnp.transpose(p1m, (0, 2, 1))
    p1nd = jnp.pad(p1m, ((0, 0), (0, 0), (0, 1)))
    idx2 = _fps(p1t, M1, M2, M2P)
    nbr2, val2, cpos2 = _nbr(p1t, p1nd, idx2[..., None], M1, R2, M2P, M1P)
    in2s = jnp.concatenate(
        [x1, p1, jnp.zeros((B, M1P, 13), F32)], axis=2)
    l2a = _pad_cols(_prep_layer(params['sa2'][0], 144), 128)
    u2 = _pre_mlp(in2s, l2a)
    g2 = _sc_gather(u2.reshape(B * M1P, 128), nbr2.reshape(-1))
    g2 = g2.reshape(B, M2P * K, 128)
    wp2 = ([l2a[0][128:132], l2a[1][128:132], l2a[2]]
           + _prep_layer(params['sa2'][1], 128)
           + _prep_layer(params['sa2'][2], 128))
    x2 = _edge_mlp(g2, cpos2, val2, wp2, (128, 256), M2P, 128)

    in2 = jnp.concatenate(
        [x2, cpos2[..., :3], jnp.zeros((B, M2P, 13), F32)], axis=2)
    wp3 = (_prep_layer(params['sa3'][0], 272)
           + _prep_layer(params['sa3'][1], 256)
           + _prep_layer(params['sa3'][2], 512))
    x3 = _sa3(in2, wp3)

    wpf3 = (_prep_layer(params['fp3'][0], 256, split=1024)
            + _prep_layer(params['fp3'][1], 256))
    h3 = _fp3(x3, x2, wpf3)

    p2m = jnp.where(jnp.arange(M2P)[None, :, None] < M2,
                    cpos2[..., :3], PPAD)
    p2t4 = jnp.transpose(jnp.pad(p2m, ((0, 0), (0, 0), (0, 1))), (0, 2, 1))
    wpf2 = (_prep_layer(params['fp2'][0], 128, split=256)
            + _prep_layer(params['fp2'][1], 256))
    h2 = _fp2(p1nd, p2t4, h3, x1, wpf2)

    p1t4 = jnp.transpose(p1nd, (0, 2, 1))
    x0e = jnp.pad(x0, ((0, 0), (0, 0), (0, 2)))
    wpf1 = (_prep_layer(params['fp1'][0], 8, split=128)
            + _prep_layer(params['fp1'][1], 128)
            + _prep_layer(params['fp1'][2], 128)
            + _prep_layer(params['head'][0], 128)
            + _prep_layer(params['head'][1], 128)
            + _prep_head_last(params['head'][2]))
    out = _fp1_head(p0nd, p1t4, h2, x0e, wpf1)
    return out.reshape(B * N0, 128)[:, :NUM_CLASSES]


# speedup vs baseline: 10.8594x; 1.0131x over previous
import functools
import numpy as np
import jax
import jax.numpy as jnp
from jax import lax
from jax.experimental import pallas as pl
from jax.experimental.pallas import tpu as pltpu
from jax.experimental.pallas import tpu_sc as plsc

B = 16
N0 = 2048
NUM_CLASSES = 13
SCALING = 2.0

M1 = 410
M1P = 512
M2 = 103
M2P = 128
K = 64
R1 = 0.2
R2 = 0.4
BIG = 1e30
PPAD = 1e9

F32 = jnp.float32
I32 = jnp.int32


def _iota(shape, dim):
    return lax.broadcasted_iota(I32, shape, dim)


def _fiota(shape, dim):
    return lax.broadcasted_iota(F32, shape, dim)


def _fold(wt, at_, bmt):
    return wt + SCALING * jnp.dot(at_, bmt, preferred_element_type=F32)


def _fps_kernel(n_real, m, pos3_ref, out_ref):
    pos3 = pos3_ref[...]
    n = pos3.shape[2]
    jn = _iota((B, n), 1)
    dmin0 = jnp.where(jn < n_real, jnp.full((B, n), jnp.inf, F32),
                      jnp.full((B, n), -jnp.inf, F32))
    idxs0 = jnp.zeros((B, out_ref.shape[1]), I32)
    last0 = pos3[:, :, 0:1]

    def body(i, state):
        dmin, idxs, lastp = state
        diff = pos3 - lastp
        dist = jnp.sum(diff * diff, axis=1)
        dmin = jnp.minimum(dmin, dist)
        mx = jnp.max(dmin, axis=1, keepdims=True)
        eq = dmin == mx
        nxt = jnp.min(jnp.where(eq, jn, n), axis=1, keepdims=True)
        idxs = jnp.where(_iota(idxs.shape, 1) == i, nxt, idxs)
        oh = (jn == nxt).astype(F32)
        lastp = jnp.sum(pos3 * oh[:, None, :], axis=2, keepdims=True)
        return dmin, idxs, lastp

    _, idxs, _ = lax.fori_loop(1, m, body, (dmin0, idxs0, last0))
    out_ref[...] = idxs


def _fps(pos3, n_real, m, mpad):
    npad = pos3.shape[2]
    return pl.pallas_call(
        functools.partial(_fps_kernel, n_real, m),
        in_specs=[pl.BlockSpec((B, 3, npad), lambda: (0, 0, 0))],
        out_specs=pl.BlockSpec((B, mpad), lambda: (0, 0)),
        out_shape=jax.ShapeDtypeStruct((B, mpad), I32),
    )(pos3)


def _nbr_kernel(n_real, r2, stride, pos3_ref, posnd_ref, idx_ref,
                nbr_ref, val_ref, cpos_ref):
    b = pl.program_id(0)
    pos3 = pos3_ref[0]
    posnd = posnd_ref[0]
    idx = idx_ref[0]
    mp = idx.shape[0]
    npad = pos3.shape[1]
    nb = npad // 128

    ohm = (_iota((mp, npad), 1) == idx).astype(F32)
    cpos = jnp.dot(ohm, posnd, preferred_element_type=F32)
    d2 = jnp.zeros((mp, npad), F32)
    for c in range(3):
        diff = cpos[:, c:c + 1] - pos3[c:c + 1, :]
        d2 = d2 + diff * diff
    maskf = (d2 <= r2).astype(F32)

    mask3 = maskf.reshape(mp, nb, 128)
    li = _iota((128, 128), 0)
    lj = _iota((128, 128), 1)
    tri_inc = (li <= lj).astype(F32)
    intra = jnp.dot(maskf.reshape(mp * nb, 128), tri_inc,
                    preferred_element_type=F32).reshape(mp, nb, 128)
    bsum = jnp.sum(mask3, axis=2)
    bi = _iota((nb, nb), 0)
    bj = _iota((nb, nb), 1)
    tri_exc = (bi < bj).astype(F32)
    base = jnp.dot(bsum, tri_exc, preferred_element_type=F32)
    crank = intra + base[:, :, None]

    cnt = jnp.sum(bsum, axis=1, keepdims=True)
    nbrf = jnp.zeros((mp, K), F32)
    tcol = _iota((mp, K), 1)
    for t in range(K):
        ind = (crank <= float(t)).astype(F32)
        c_t = jnp.sum(jnp.sum(ind, axis=2), axis=1, keepdims=True)
        nbrf = jnp.where(tcol == t, c_t, nbrf)
    nbrf = jnp.minimum(nbrf, float(n_real - 1))
    validf = (tcol.astype(F32) < jnp.minimum(cnt, float(K))).astype(F32)

    nbr_ref[0] = nbrf.astype(I32) + b * stride
    val_ref[0] = validf
    cpos_ref[0] = cpos


def _nbr(pos3, posnd, idx3, n_real, r, mpad, stride):
    npad = pos3.shape[2]
    return pl.pallas_call(
        functools.partial(_nbr_kernel, n_real, r * r, stride),
        grid=(B,),
        in_specs=[
            pl.BlockSpec((1, 3, npad), lambda b: (b, 0, 0)),
            pl.BlockSpec((1, npad, 4), lambda b: (b, 0, 0)),
            pl.BlockSpec((1, mpad, 1), lambda b: (b, 0, 0)),
        ],
        out_specs=[
            pl.BlockSpec((1, mpad, K), lambda b: (b, 0, 0)),
            pl.BlockSpec((1, mpad, K), lambda b: (b, 0, 0)),
            pl.BlockSpec((1, mpad, 4), lambda b: (b, 0, 0)),
        ],
        out_shape=[
            jax.ShapeDtypeStruct((B, mpad, K), I32),
            jax.ShapeDtypeStruct((B, mpad, K), F32),
            jax.ShapeDtypeStruct((B, mpad, 4), F32),
        ],
    )(pos3, posnd, idx3)


def _sc_gather(table, idx, group=2):
    e = idx.shape[0]
    d = table.shape[1]
    nw = 32
    pw128 = e // (nw * 128)
    npair = pw128 // (2 * group)
    idx2 = idx.reshape(e // 128, 128)
    mesh = plsc.VectorSubcoreMesh(core_axis_name="c", subcore_axis_name="s")

    @functools.partial(
        pl.kernel,
        out_type=jax.ShapeDtypeStruct((e, d), F32),
        mesh=mesh,
        scratch_types=[
            pltpu.VMEM((pw128, 128), I32),
            pltpu.VMEM((group * 128, d), F32),
            pltpu.VMEM((group * 128, d), F32),
            pltpu.SemaphoreType.DMA,
            pltpu.SemaphoreType.DMA,
            pltpu.SemaphoreType.DMA,
        ],
    )
    def k(table_hbm, idx_hbm, out_hbm, idx_v, rows0, rows1,
          gsem, wsem0, wsem1):
        wid = lax.axis_index("s") * 2 + lax.axis_index("c")
        row0 = wid * pw128
        pltpu.sync_copy(idx_hbm.at[pl.ds(row0, pw128)], idx_v)

        def step(buf, wsem, g, first):
            @pl.when(jnp.logical_not(first))
            def _():
                pltpu.make_async_copy(
                    buf, out_hbm.at[pl.ds(row0 * 128, group * 128)],
                    wsem).wait()
            copies = []
            for j in range(group):
                copies.append(pltpu.async_copy(
                    table_hbm.at[idx_v.at[g * group + j]],
                    buf.at[pl.ds(j * 128, 128)], gsem))
            for cp in copies:
                cp.wait()
            pltpu.async_copy(
                buf, out_hbm.at[pl.ds((row0 + g * group) * 128,
                                      group * 128)], wsem)

        def body(p, carry):
            step(rows0, wsem0, p * 2, p == 0)
            step(rows1, wsem1, p * 2 + 1, p == 0)
            return carry

        lax.fori_loop(0, npair, body, 0)
        for buf, wsem in ((rows0, wsem0), (rows1, wsem1)):
            pltpu.make_async_copy(
                buf, out_hbm.at[pl.ds(row0 * 128, group * 128)], wsem).wait()

    return k(table, idx2)


def _pre_kernel(in_ref, w_ref, a_ref, p_ref, b_ref, out_ref):
    wc = _fold(w_ref[...], a_ref[...], p_ref[...])
    out_ref[0] = (jnp.dot(in_ref[0], wc, preferred_element_type=F32)
                  + b_ref[...])


def _pre_mlp(inp, wp):
    _, np_, d = inp.shape

    def wspec(shape):
        return pl.BlockSpec(shape, lambda b: tuple(0 for _ in shape))

    return pl.pallas_call(
        _pre_kernel,
        grid=(B,),
        in_specs=[pl.BlockSpec((1, np_, d), lambda b: (b, 0, 0)),
                  wspec((d, 128)), wspec((d, 8)), wspec((8, 128)),
                  wspec((1, 128))],
        out_specs=pl.BlockSpec((1, np_, 128), lambda b: (b, 0, 0)),
        out_shape=jax.ShapeDtypeStruct((B, np_, 128), F32),
    )(inp, *wp)


def _edge_kernel(g_ref, cp_ref, val_ref,
                 wp_ref, ap_ref, pm_ref,
                 w2_ref, a2_ref, p2_ref, b2_ref,
                 w3_ref, a3_ref, p3_ref, b3_ref, out_ref):
    g2 = g_ref[0]
    cp = cp_ref[0]
    vmask = val_ref[0]
    mc = cp.shape[0]
    c3 = out_ref.shape[2]

    w1p = _fold(wp_ref[...], ap_ref[...], pm_ref[...])
    w2c = _fold(w2_ref[...], a2_ref[...], p2_ref[...])
    w3c = _fold(w3_ref[...], a3_ref[...], p3_ref[...])

    ccon = jnp.dot(cp, w1p, preferred_element_type=F32)
    h = g2.reshape(mc, K, 128) - ccon[:, None, :]
    h = jnp.maximum(h, 0.0).reshape(mc * K, 128)
    h = jnp.maximum(jnp.dot(h, w2c, preferred_element_type=F32)
                    + b2_ref[...], 0.0)
    h = jnp.dot(h, w3c, preferred_element_type=F32) + b3_ref[...]
    h = h.reshape(mc, K, c3)
    h = jnp.where(vmask[:, :, None] > 0.0, h, -BIG)
    mx = jnp.max(h, axis=1)
    out_ref[0] = jnp.where(mx > -BIG * 0.5, mx, 0.0)


def _edge_mlp(g3, cpos, validf, wp, couts, mpad, mc):
    grid_m = mpad // mc
    c2, c3 = couts

    def wspec(shape):
        return pl.BlockSpec(shape, lambda b, i: tuple(0 for _ in shape))

    return pl.pallas_call(
        _edge_kernel,
        grid=(B, grid_m),
        in_specs=[
            pl.BlockSpec((1, mc * K, 128), lambda b, i: (b, i, 0)),
            pl.BlockSpec((1, mc, 4), lambda b, i: (b, i, 0)),
            pl.BlockSpec((1, mc, K), lambda b, i: (b, i, 0)),
            wspec((4, 128)), wspec((4, 8)), wspec((8, 128)),
            wspec((128, c2)), wspec((128, 8)), wspec((8, c2)), wspec((1, c2)),
            wspec((c2, c3)), wspec((c2, 8)), wspec((8, c3)), wspec((1, c3)),
        ],
        out_specs=pl.BlockSpec((1, mc, c3), lambda b, i: (b, i, 0)),
        out_shape=jax.ShapeDtypeStruct((B, mpad, c3), F32),
    )(g3, cpos, validf, *wp)


def _sa3_kernel(w_refs, in_ref, out_ref):
    (w1, a1, p1, b1, w2, a2, p2, b2, w3, a3, p3, b3) = w_refs
    x = in_ref[0]
    h = jnp.maximum(jnp.dot(x, _fold(w1[...], a1[...], p1[...]),
                            preferred_element_type=F32) + b1[...], 0.0)
    h = jnp.maximum(jnp.dot(h, _fold(w2[...], a2[...], p2[...]),
                            preferred_element_type=F32) + b2[...], 0.0)
    h = jnp.dot(h, _fold(w3[...], a3[...], p3[...]),
                preferred_element_type=F32) + b3[...]
    h = jnp.where(_iota(h.shape, 0) < M2, h, -BIG)
    out_ref[0] = jnp.max(h, axis=0, keepdims=True)


def _sa3(in2, wp):
    cin = in2.shape[2]

    def body(*refs):
        _sa3_kernel(refs[1:13], refs[0], refs[13])

    def wspec(shape):
        return pl.BlockSpec(shape, lambda b: tuple(0 for _ in shape))

    shapes = [(cin, 256), (cin, 8), (8, 256), (1, 256),
              (256, 512), (256, 8), (8, 512), (1, 512),
              (512, 1024), (512, 8), (8, 1024), (1, 1024)]
    return pl.pallas_call(
        body,
        grid=(B,),
        in_specs=[pl.BlockSpec((1, M2P, cin), lambda b: (b, 0, 0))]
        + [wspec(s) for s in shapes],
        out_specs=pl.BlockSpec((1, 1, 1024), lambda b: (b, 0, 0)),
        out_shape=jax.ShapeDtypeStruct((B, 1, 1024), F32),
    )(in2, *wp)


def _fp3_kernel(x3_ref, x2_ref, w1a_ref, a1a_ref, w1b_ref, a1b_ref,
                p1_ref, b1_ref, w2_ref, a2_ref, p2_ref, b2_ref, out_ref):
    x3 = x3_ref[0]
    x2 = x2_ref[0]
    w1a = _fold(w1a_ref[...], a1a_ref[...], p1_ref[...])
    row = jnp.dot(x3, w1a, preferred_element_type=F32) + b1_ref[...]
    h = jnp.maximum(jnp.dot(x2, _fold(w1b_ref[...], a1b_ref[...], p1_ref[...]),
                            preferred_element_type=F32) + row, 0.0)
    w2c = _fold(w2_ref[...], a2_ref[...], p2_ref[...])
    out_ref[0] = jnp.dot(h, w2c, preferred_element_type=F32) + b2_ref[...]


def _fp3(x3, x2, wp):
    def wspec(shape):
        return pl.BlockSpec(shape, lambda b: tuple(0 for _ in shape))

    shapes = [(1024, 256), (1024, 8), (256, 256), (256, 8), (8, 256), (1, 256),
              (256, 256), (256, 8), (8, 256), (1, 256)]
    return pl.pallas_call(
        _fp3_kernel,
        grid=(B,),
        in_specs=[pl.BlockSpec((1, 1, 1024), lambda b: (b, 0, 0)),
                  pl.BlockSpec((1, M2P, 256), lambda b: (b, 0, 0))]
        + [wspec(s) for s in shapes],
        out_specs=pl.BlockSpec((1, M2P, 256), lambda b: (b, 0, 0)),
        out_shape=jax.ShapeDtypeStruct((B, M2P, 256), F32),
    )(x3, x2, *wp)


def _interp_weights(pdst, psrct, n_src_real):
    nd = pdst.shape[0]
    nsp = psrct.shape[1]
    d2 = jnp.zeros((nd, nsp), F32)
    for c in range(3):
        diff = pdst[:, c:c + 1] - psrct[c:c + 1, :]
        d2 = d2 + diff * diff
    jcol = _iota((nd, nsp), 1)
    d2m = jnp.where(jcol < n_src_real, d2, BIG)
    wacc = jnp.zeros((nd, nsp), F32)
    sel = jnp.zeros((nd, nsp), F32)
    for _ in range(3):
        cur = jnp.where(sel > 0.0, BIG, d2m)
        v = jnp.min(cur, axis=1, keepdims=True)
        eq = cur == v
        jm = jnp.min(jnp.where(eq, jcol, nsp), axis=1, keepdims=True)
        oh = (jcol == jm).astype(F32)
        w = 1.0 / jnp.maximum(v, 1e-16)
        wacc = wacc + oh * w
        sel = sel + oh
    return wacc / jnp.sum(wacc, axis=1, keepdims=True)


def _fp2_kernel(pdst_ref, psrct_ref, h3_ref, x1_ref,
                w1a_ref, a1a_ref, w1b_ref, a1b_ref, p1_ref, b1_ref,
                w2_ref, a2_ref, p2_ref, b2_ref, out_ref):
    wn = _interp_weights(pdst_ref[0], psrct_ref[0], M2)
    interp = jnp.dot(wn, h3_ref[0], preferred_element_type=F32)
    h = (jnp.dot(interp, _fold(w1a_ref[...], a1a_ref[...], p1_ref[...]),
                 preferred_element_type=F32)
         + jnp.dot(x1_ref[0], _fold(w1b_ref[...], a1b_ref[...], p1_ref[...]),
                   preferred_element_type=F32) + b1_ref[...])
    h = jnp.maximum(h, 0.0)
    out_ref[0] = (jnp.dot(h, _fold(w2_ref[...], a2_ref[...], p2_ref[...]),
                          preferred_element_type=F32) + b2_ref[...])


def _fp2(cpos1, psrc2t, h3, x1, wp):
    def wspec(shape):
        return pl.BlockSpec(shape, lambda b: tuple(0 for _ in shape))

    shapes = [(256, 256), (256, 8), (128, 256), (128, 8), (8, 256), (1, 256),
              (256, 128), (256, 8), (8, 128), (1, 128)]
    return pl.pallas_call(
        _fp2_kernel,
        grid=(B,),
        in_specs=[pl.BlockSpec((1, M1P, 4), lambda b: (b, 0, 0)),
                  pl.BlockSpec((1, 4, M2P), lambda b: (b, 0, 0)),
                  pl.BlockSpec((1, M2P, 256), lambda b: (b, 0, 0)),
                  pl.BlockSpec((1, M1P, 128), lambda b: (b, 0, 0))]
        + [wspec(s) for s in shapes],
        out_specs=pl.BlockSpec((1, M1P, 128), lambda b: (b, 0, 0)),
        out_shape=jax.ShapeDtypeStruct((B, M1P, 128), F32),
    )(cpos1, psrc2t, h3, x1, *wp)


def _fp1_head_kernel(p0_ref, psrc1t_ref, h2_ref, x0_ref, wrefs, out_ref):
    (w1a, a1a, w1b, a1b, p1, b1, w2, a2, p2, b2, w3, a3, p3, b3,
     hw1, ha1, hp1, hb1, hw2, ha2, hp2, hb2, hw3, ha3, hp3, hb3) = wrefs
    wn = _interp_weights(p0_ref[0], psrc1t_ref[0], M1)
    interp = jnp.dot(wn, h2_ref[0], preferred_element_type=F32)
    h = (jnp.dot(interp, _fold(w1a[...], a1a[...], p1[...]),
                 preferred_element_type=F32)
         + jnp.dot(x0_ref[0], _fold(w1b[...], a1b[...], p1[...]),
                   preferred_element_type=F32) + b1[...])
    h = jnp.maximum(h, 0.0)
    h = jnp.maximum(jnp.dot(h, _fold(w2[...], a2[...], p2[...]),
                            preferred_element_type=F32) + b2[...], 0.0)
    h = jnp.dot(h, _fold(w3[...], a3[...], p3[...]),
                preferred_element_type=F32) + b3[...]
    h = jnp.maximum(jnp.dot(h, _fold(hw1[...], ha1[...], hp1[...]),
                            preferred_element_type=F32) + hb1[...], 0.0)
    h = jnp.maximum(jnp.dot(h, _fold(hw2[...], ha2[...], hp2[...]),
                            preferred_element_type=F32) + hb2[...], 0.0)
    o = jnp.dot(h, _fold(hw3[...], ha3[...], hp3[...]),
                preferred_element_type=F32) + hb3[...]
    mx = jnp.max(o, axis=-1, keepdims=True)
    e = jnp.where(o > -BIG * 0.5, jnp.exp(o - mx), 0.0)
    lse = jnp.log(jnp.sum(e, axis=-1, keepdims=True)) + mx
    out_ref[0] = o - lse


def _fp1_head(p0nd, psrc1t, h2, x0e, wp):
    def wspec(shape):
        return pl.BlockSpec(shape, lambda b: tuple(0 for _ in shape))

    shapes = [(128, 128), (128, 8), (8, 128), (8, 8), (8, 128), (1, 128),
              (128, 128), (128, 8), (8, 128), (1, 128),
              (128, 128), (128, 8), (8, 128), (1, 128),
              (128, 128), (128, 8), (8, 128), (1, 128),
              (128, 128), (128, 8), (8, 128), (1, 128),
              (128, 128), (128, 8), (8, 128), (1, 128)]

    def body(*refs):
        _fp1_head_kernel(refs[0], refs[1], refs[2], refs[3],
                         refs[4:30], refs[30])

    return pl.pallas_call(
        body,
        grid=(B,),
        in_specs=[pl.BlockSpec((1, N0, 4), lambda b: (b, 0, 0)),
                  pl.BlockSpec((1, 4, M1P), lambda b: (b, 0, 0)),
                  pl.BlockSpec((1, M1P, 128), lambda b: (b, 0, 0)),
                  pl.BlockSpec((1, N0, 8), lambda b: (b, 0, 0))]
        + [wspec(s) for s in shapes],
        out_specs=pl.BlockSpec((1, N0, 128), lambda b: (b, 0, 0)),
        out_shape=jax.ShapeDtypeStruct((B, N0, 128), F32),
    )(p0nd, psrc1t, h2, x0e, *wp)


def _prep_layer(p, fin_pad, split=None):
    w, a, bm, bias = p['W'], p['A'], p['Bm'], p['b']
    fout, fin = w.shape
    wt = w.T
    at = a.T
    bmt = bm.T
    b2 = bias[None, :]
    if split is None:
        pad = fin_pad - fin
        if pad:
            wt = jnp.pad(wt, ((0, pad), (0, 0)))
            at = jnp.pad(at, ((0, pad), (0, 0)))
        return [wt, at, bmt, b2]
    s0 = split
    wta, wtb = wt[:s0], wt[s0:]
    ata, atb = at[:s0], at[s0:]
    padb = fin_pad - (fin - s0)
    if padb:
        wtb = jnp.pad(wtb, ((0, padb), (0, 0)))
        atb = jnp.pad(atb, ((0, padb), (0, 0)))
    return [wta, ata, wtb, atb, bmt, b2]


def _pad_cols(arrs, cout):
    wt, at, bmt, b2 = arrs
    pad = cout - wt.shape[1]
    if pad:
        wt = jnp.pad(wt, ((0, 0), (0, pad)))
        bmt = jnp.pad(bmt, ((0, 0), (0, pad)))
        b2 = jnp.pad(b2, ((0, 0), (0, pad)))
    return [wt, at, bmt, b2]


def _prep_head_last(p):
    w, a, bm, bias = p['W'], p['A'], p['Bm'], p['b']
    wt = jnp.pad(w.T, ((0, 0), (0, 128 - NUM_CLASSES)))
    bmt = jnp.pad(bm.T, ((0, 0), (0, 128 - NUM_CLASSES)))
    b2 = jnp.full((1, 128), -jnp.inf, F32).at[0, :NUM_CLASSES].set(bias)
    return [wt, a.T, bmt, b2]


def kernel(x, pos, batch, params):
    x0 = x.reshape(B, N0, 6)
    p0 = pos.reshape(B, N0, 3)
    p0t = jnp.transpose(p0, (0, 2, 1))
    p0nd = jnp.pad(p0, ((0, 0), (0, 0), (0, 1)))

    idx1 = _fps(p0t, N0, M1, M1P)
    nbr1, val1, cpos1 = _nbr(p0t, p0nd, idx1[..., None], N0, R1, M1P, N0)
    in1 = jnp.concatenate(
        [x0, p0, jnp.zeros((B, N0, 7), F32)], axis=2)
    l1a = _pad_cols(_prep_layer(params['sa1'][0], 16), 128)
    u1 = _pre_mlp(in1, l1a)
    g1 = _sc_gather(u1.reshape(B * N0, 128), nbr1.reshape(-1))
    g1 = g1.reshape(B, M1P * K, 128)
    wp1 = ([l1a[0][6:10], l1a[1][6:10], l1a[2]]
           + _prep_layer(params['sa1'][1], 128)
           + _prep_layer(params['sa1'][2], 64))
    x1 = _edge_mlp(g1, cpos1, val1, wp1, (64, 128), M1P, 128)

    p1 = cpos1[..., :3]
    jrow = jnp.arange(M1P)[None, :, None]
    p1m = jnp.where(jrow < M1, p1, PPAD)
    p1t = jnp.transpose(p1m, (0, 2, 1))
    p1nd = jnp.pad(p1m, ((0, 0), (0, 0), (0, 1)))
    idx2 = _fps(p1t, M1, M2, M2P)
    nbr2, val2, cpos2 = _nbr(p1t, p1nd, idx2[..., None], M1, R2, M2P, M1P)
    in2s = jnp.concatenate(
        [x1, p1, jnp.zeros((B, M1P, 13), F32)], axis=2)
    l2a = _pad_cols(_prep_layer(params['sa2'][0], 144), 128)
    u2 = _pre_mlp(in2s, l2a)
    g2 = _sc_gather(u2.reshape(B * M1P, 128), nbr2.reshape(-1))
    g2 = g2.reshape(B, M2P * K, 128)
    wp2 = ([l2a[0][128:132], l2a[1][128:132], l2a[2]]
           + _prep_layer(params['sa2'][1], 128)
           + _prep_layer(params['sa2'][2], 128))
    x2 = _edge_mlp(g2, cpos2, val2, wp2, (128, 256), M2P, 128)

    in2 = jnp.concatenate(
        [x2, cpos2[..., :3], jnp.zeros((B, M2P, 13), F32)], axis=2)
    wp3 = (_prep_layer(params['sa3'][0], 272)
           + _prep_layer(params['sa3'][1], 256)
           + _prep_layer(params['sa3'][2], 512))
    x3 = _sa3(in2, wp3)

    wpf3 = (_prep_layer(params['fp3'][0], 256, split=1024)
            + _prep_layer(params['fp3'][1], 256))
    h3 = _fp3(x3, x2, wpf3)

    p2m = jnp.where(jnp.arange(M2P)[None, :, None] < M2,
                    cpos2[..., :3], PPAD)
    p2t4 = jnp.transpose(jnp.pad(p2m, ((0, 0), (0, 0), (0, 1))), (0, 2, 1))
    wpf2 = (_prep_layer(params['fp2'][0], 128, split=256)
            + _prep_layer(params['fp2'][1], 256))
    h2 = _fp2(p1nd, p2t4, h3, x1, wpf2)

    p1t4 = jnp.transpose(p1nd, (0, 2, 1))
    x0e = jnp.pad(x0, ((0, 0), (0, 0), (0, 2)))
    wpf1 = (_prep_layer(params['fp1'][0], 8, split=128)
            + _prep_layer(params['fp1'][1], 128)
            + _prep_layer(params['fp1'][2], 128)
            + _prep_layer(params['head'][0], 128)
            + _prep_layer(params['head'][1], 128)
            + _prep_head_last(params['head'][2]))
    out = _fp1_head(p0nd, p1t4, h2, x0e, wpf1)
    return out.reshape(B * N0, 128)[:, :NUM_CLASSES]
